# Initial kernel scaffold; baseline (speedup 1.0000x reference)
import jax
import jax.numpy as jnp
from jax.experimental import pallas as pl


def kernel(subject_embeddings, object_embeddings, relations, relation_weight):
    raise NotImplementedError("write your pallas kernel here")


# speedup vs baseline: 1.1626x; 1.1626x over previous
import jax
import jax.numpy as jnp
from jax import lax
from jax.experimental import pallas as pl
from jax.experimental.pallas import tpu as pltpu
from jax.experimental.pallas import tpu_sc as plsc

_B = 16384
_D = 64
_EPS = 1e-6
_NC = 2
_NS = 16
_NW = _NC * _NS
_BPW = _B // _NW
_L = 16


def _sc_body(subj_hbm, obj_hbm, rel_hbm, relw_hbm, out_hbm,
             idx_v, rel_v, s_v, o_v, sc_v, sem):
    wid = lax.axis_index("s") * _NC + lax.axis_index("c")
    base = wid * _BPW

    pltpu.sync_copy(rel_hbm.at[pl.ds(base, _BPW)], idx_v)
    gather = pltpu.async_copy(relw_hbm.at[idx_v], rel_v, sem)
    pltpu.sync_copy(subj_hbm.at[pl.ds(base, _BPW)], s_v)
    pltpu.sync_copy(obj_hbm.at[pl.ds(base, _BPW)], o_v)
    gather.wait()

    lane = lax.iota(jnp.int32, _L)

    def group(g, carry):
        rows_base = g * _L
        scores = jnp.zeros((_L,), jnp.float32)
        for j in range(_L):
            r = rows_base + j
            acc = jnp.zeros((_L,), jnp.float32)
            for c in range(_D // _L):
                sl = pl.ds(c * _L, _L)
                d = s_v[r, sl] + rel_v[r, sl] - o_v[r, sl] + _EPS
                acc = acc + d * d
            for s in (8, 4, 2, 1):
                acc = acc + acc.at[lane ^ s].get(mode="promise_in_bounds")
            scores = jnp.where(lane == j, acc, scores)
        x = jnp.maximum(scores, 1e-35)
        xi = lax.bitcast_convert_type(x, jnp.int32)
        y = lax.bitcast_convert_type(jnp.int32(0x5F3759DF) - (xi >> 1),
                                     jnp.float32)
        for _ in range(3):
            y = y * (1.5 - 0.5 * x * y * y)
        sc_v[pl.ds(g * _L, _L)] = x * y
        return carry

    lax.fori_loop(0, _BPW // _L, group, 0)
    pltpu.sync_copy(sc_v, out_hbm.at[pl.ds(base, _BPW)])


def kernel(subject_embeddings, object_embeddings, relations, relation_weight):
    relations = relations.astype(jnp.int32)
    mesh = plsc.VectorSubcoreMesh(core_axis_name="c", subcore_axis_name="s")
    k = pl.kernel(
        _sc_body,
        mesh=mesh,
        compiler_params=pltpu.CompilerParams(use_tc_tiling_on_sc=False),
        out_type=jax.ShapeDtypeStruct((_B,), jnp.float32),
        scratch_types=[
            pltpu.VMEM((_BPW,), jnp.int32),
            pltpu.VMEM((_BPW, _D), jnp.float32),
            pltpu.VMEM((_BPW, _D), jnp.float32),
            pltpu.VMEM((_BPW, _D), jnp.float32),
            pltpu.VMEM((_BPW,), jnp.float32),
            pltpu.SemaphoreType.DMA,
        ],
    )
    return k(subject_embeddings, object_embeddings, relations, relation_weight)


# speedup vs baseline: 1.3898x; 1.1954x over previous
import jax
import jax.numpy as jnp
from jax import lax
from jax.experimental import pallas as pl
from jax.experimental.pallas import tpu as pltpu
from jax.experimental.pallas import tpu_sc as plsc

_B = 16384
_D = 64
_R = 1000
_EPS = 1e-6
_NC = 2
_NS = 16
_NW = _NC * _NS
_BPW = _B // _NW
_CH = 128
_NCH = _BPW // _CH
_L = 16


def _sc_body(subj_hbm, obj_hbm, rel_hbm, relw_hbm, out_hbm,
             idx_v, tab_v, s_v, o_v, sc_v, sem_t, sem_s0, sem_s1,
             sem_o0, sem_o1):
    wid = lax.axis_index("s") * _NC + lax.axis_index("c")
    base = wid * _BPW

    tab = pltpu.async_copy(relw_hbm, tab_v, sem_t)
    pltpu.sync_copy(rel_hbm.at[pl.ds(base, _BPW)], idx_v)

    lane = lax.iota(jnp.int32, _L)

    def stage(ch):
        par = ch % 2
        cbase = base + ch * _CH
        hs = pltpu.async_copy(subj_hbm.at[pl.ds(cbase, _CH)], s_v.at[par],
                              sem_s0 if par == 0 else sem_s1)
        ho = pltpu.async_copy(obj_hbm.at[pl.ds(cbase, _CH)], o_v.at[par],
                              sem_o0 if par == 0 else sem_o1)
        return hs, ho

    pend = stage(0)
    tab.wait()

    for ch in range(_NCH):
        par = ch % 2
        hs, ho = pend
        if ch + 1 < _NCH:
            pend = stage(ch + 1)
        hs.wait()
        ho.wait()

        def group(g, carry):
            rows_base = g * _L
            scores = jnp.zeros((_L,), jnp.float32)
            tvec = idx_v[pl.ds(ch * _CH + rows_base, _L)]
            tbase = tvec * _D
            for j in range(_L):
                r = rows_base + j
                off = tbase[j]
                acc = jnp.zeros((_L,), jnp.float32)
                for c in range(_D // _L):
                    sl = pl.ds(c * _L, _L)
                    rel = tab_v[pl.ds(off + c * _L, _L)]
                    d = s_v[par, r, sl] + rel - o_v[par, r, sl] + _EPS
                    acc = acc + d * d
                for s in (8, 4, 2, 1):
                    acc = acc + acc.at[lane ^ s].get(mode="promise_in_bounds")
                scores = jnp.where(lane == j, acc, scores)
            x = jnp.maximum(scores, 1e-35)
            xi = lax.bitcast_convert_type(x, jnp.int32)
            y = lax.bitcast_convert_type(jnp.int32(0x5F3759DF) - (xi >> 1),
                                         jnp.float32)
            for _ in range(3):
                y = y * (1.5 - 0.5 * x * y * y)
            sc_v[pl.ds(ch * _CH + rows_base, _L)] = x * y
            return carry

        lax.fori_loop(0, _CH // _L, group, 0)

    pltpu.sync_copy(sc_v, out_hbm.at[pl.ds(base, _BPW)])


def kernel(subject_embeddings, object_embeddings, relations, relation_weight):
    relations = relations.astype(jnp.int32)
    relw_flat = relation_weight.reshape(-1)
    mesh = plsc.VectorSubcoreMesh(core_axis_name="c", subcore_axis_name="s")
    k = pl.kernel(
        _sc_body,
        mesh=mesh,
        out_type=jax.ShapeDtypeStruct((_B,), jnp.float32),
        scratch_types=[
            pltpu.VMEM((_BPW,), jnp.int32),
            pltpu.VMEM((_R * _D,), jnp.float32),
            pltpu.VMEM((2, _CH, _D), jnp.float32),
            pltpu.VMEM((2, _CH, _D), jnp.float32),
            pltpu.VMEM((_BPW,), jnp.float32),
            pltpu.SemaphoreType.DMA,
            pltpu.SemaphoreType.DMA,
            pltpu.SemaphoreType.DMA,
            pltpu.SemaphoreType.DMA,
            pltpu.SemaphoreType.DMA,
        ],
    )
    return k(subject_embeddings, object_embeddings, relations, relw_flat)


# speedup vs baseline: 1.3913x; 1.0010x over previous
import jax
import jax.numpy as jnp
from jax import lax
from jax.experimental import pallas as pl
from jax.experimental.pallas import tpu as pltpu
from jax.experimental.pallas import tpu_sc as plsc

_B = 16384
_D = 64
_R = 1000
_EPS = 1e-6
_NC = 2
_NS = 16
_NW = _NC * _NS
_BPW = _B // _NW
_CH = 128
_NCH = _BPW // _CH
_L = 16


def _sc_body(subj_hbm, obj_hbm, rel_hbm, relw_hbm, out_hbm,
             idx_v, tab_v, s_v, o_v, sc_v, sem_t, sem_s0, sem_s1,
             sem_o0, sem_o1):
    wid = lax.axis_index("s") * _NC + lax.axis_index("c")
    base = wid * _BPW

    tab = pltpu.async_copy(relw_hbm, tab_v, sem_t)
    pltpu.sync_copy(rel_hbm.at[pl.ds(base, _BPW)], idx_v)

    lane = lax.iota(jnp.int32, _L)

    def stage(ch):
        par = ch % 2
        cbase = base + ch * _CH
        hs = pltpu.async_copy(subj_hbm.at[pl.ds(cbase, _CH)], s_v.at[par],
                              sem_s0 if par == 0 else sem_s1)
        ho = pltpu.async_copy(obj_hbm.at[pl.ds(cbase, _CH)], o_v.at[par],
                              sem_o0 if par == 0 else sem_o1)
        return hs, ho

    pend = stage(0)
    tab.wait()

    for ch in range(_NCH):
        par = ch % 2
        hs, ho = pend
        if ch + 1 < _NCH:
            pend = stage(ch + 1)
        hs.wait()
        ho.wait()

        def group(g, carry):
            rows_base = g * _L
            scores = jnp.zeros((_L,), jnp.float32)
            tvec = idx_v[pl.ds(ch * _CH + rows_base, _L)]
            tbase = tvec * _D
            for j in range(_L):
                r = rows_base + j
                off = tbase[j]
                acc = jnp.zeros((_L,), jnp.float32)
                for c in range(_D // _L):
                    sl = pl.ds(c * _L, _L)
                    rel = tab_v[pl.ds(off + c * _L, _L)]
                    d = s_v[par, r, sl] + rel - o_v[par, r, sl] + _EPS
                    acc = acc + d * d
                for s in (8, 4, 2, 1):
                    acc = acc + acc.at[lane ^ s].get(mode="promise_in_bounds")
                scores = jnp.where(lane == j, acc, scores)
            x = jnp.maximum(scores, 1e-35)
            xi = lax.bitcast_convert_type(x, jnp.int32)
            y = lax.bitcast_convert_type(jnp.int32(0x5F3759DF) - (xi >> 1),
                                         jnp.float32)
            for _ in range(3):
                y = y * (1.5 - 0.5 * x * y * y)
            sc_v[pl.ds(ch * _CH + rows_base, _L)] = x * y
            return carry

        lax.fori_loop(0, _CH // _L, group, 0)

    pltpu.sync_copy(sc_v, out_hbm.at[pl.ds(base, _BPW)])


def kernel(subject_embeddings, object_embeddings, relations, relation_weight):
    relations = relations.astype(jnp.int32)
    relw_flat = relation_weight.reshape(-1)
    mesh = plsc.VectorSubcoreMesh(core_axis_name="c", subcore_axis_name="s")
    k = pl.kernel(
        _sc_body,
        mesh=mesh,
        compiler_params=pltpu.CompilerParams(use_tc_tiling_on_sc=True),
        out_type=jax.ShapeDtypeStruct((_B,), jnp.float32),
        scratch_types=[
            pltpu.VMEM((_BPW,), jnp.int32),
            pltpu.VMEM((_R * _D,), jnp.float32),
            pltpu.VMEM((2, _CH, _D), jnp.float32),
            pltpu.VMEM((2, _CH, _D), jnp.float32),
            pltpu.VMEM((_BPW,), jnp.float32),
            pltpu.SemaphoreType.DMA,
            pltpu.SemaphoreType.DMA,
            pltpu.SemaphoreType.DMA,
            pltpu.SemaphoreType.DMA,
            pltpu.SemaphoreType.DMA,
        ],
    )
    return k(subject_embeddings, object_embeddings, relations, relw_flat)
